# baseline (speedup 1.0000x reference)
import jax
import jax.numpy as jnp
from jax.experimental import pallas as pl
from jax.experimental.pallas import tpu as pltpu


_BLK = 256


def _trim_body(nblk, m_ref, v_ref, x_hbm, mo_ref, vo_ref, xo_hbm,
               ml_s, xin, xout, zbuf, in_sem, out_sem):
    i = pl.program_id(0)

    @pl.when(i == 0)
    def _prologue():
        pltpu.make_async_copy(
            x_hbm.at[:, pl.ds(0, _BLK)], xin.at[0], in_sem.at[0]).start()
        counts = jnp.sum((m_ref[...] != 0).astype(jnp.int32), axis=1)
        ml0 = jnp.maximum(jnp.max(counts), 1)
        ml_s[0] = ml0
        col = jax.lax.broadcasted_iota(jnp.int32, (1, v_ref.shape[1]), 1)
        keep = col < ml0
        vo_ref[...] = jnp.where(keep, v_ref[...], 0.0)
        mo_ref[...] = jnp.where(keep & (m_ref[...] != 0), 1, 0).astype(jnp.int32)
        zbuf[...] = jnp.zeros_like(zbuf)

    ml = ml_s[0]
    slot = jax.lax.rem(i, 2)

    nxt = i + 1
    nslot = jax.lax.rem(nxt, 2)

    @pl.when((nxt < nblk) & (nxt * _BLK < ml))
    def _start_next():
        pltpu.make_async_copy(
            x_hbm.at[:, pl.ds(nxt * _BLK, _BLK)], xin.at[nslot],
            in_sem.at[nslot]).start()

    prev = i - 2

    @pl.when(prev >= 0)
    def _wait_prev_out():
        @pl.when(prev * _BLK < ml)
        def _():
            pltpu.make_async_copy(
                xout.at[slot], xo_hbm.at[:, pl.ds(prev * _BLK, _BLK)],
                out_sem.at[slot]).wait()

        @pl.when(prev * _BLK >= ml)
        def _():
            pltpu.make_async_copy(
                zbuf, xo_hbm.at[:, pl.ds(prev * _BLK, _BLK)],
                out_sem.at[slot]).wait()

    needed = i * _BLK < ml

    @pl.when(needed)
    def _compute():
        pltpu.make_async_copy(
            x_hbm.at[:, pl.ds(i * _BLK, _BLK)], xin.at[slot],
            in_sem.at[slot]).wait()
        col = i * _BLK + jax.lax.broadcasted_iota(jnp.int32, (1, _BLK), 1)
        xout[slot] = jnp.where(col < ml, xin[slot], 0.0)
        pltpu.make_async_copy(
            xout.at[slot], xo_hbm.at[:, pl.ds(i * _BLK, _BLK)],
            out_sem.at[slot]).start()

    @pl.when(jnp.logical_not(needed))
    def _zero_out():
        pltpu.make_async_copy(
            zbuf, xo_hbm.at[:, pl.ds(i * _BLK, _BLK)],
            out_sem.at[slot]).start()

    @pl.when(i == nblk - 1)
    def _epilogue():
        for d in (nblk - 2, nblk - 1):
            s = d % 2

            @pl.when(d * _BLK < ml)
            def _():
                pltpu.make_async_copy(
                    xout.at[s], xo_hbm.at[:, pl.ds(d * _BLK, _BLK)],
                    out_sem.at[s]).wait()

            @pl.when(d * _BLK >= ml)
            def _():
                pltpu.make_async_copy(
                    zbuf, xo_hbm.at[:, pl.ds(d * _BLK, _BLK)],
                    out_sem.at[s]).wait()


def kernel(x, v, mask):
    B, C, P = x.shape
    CV = v.shape[1]
    R = B * C
    nblk = P // _BLK
    xr = x.reshape(R, P)
    vr = v.reshape(B * CV, P)
    mr = mask.reshape(B, P)

    import functools
    body = functools.partial(_trim_body, nblk)

    mo, vo, xo = pl.pallas_call(
        body,
        grid=(nblk,),
        in_specs=[
            pl.BlockSpec((B, P), lambda i: (0, 0)),
            pl.BlockSpec((B * CV, P), lambda i: (0, 0)),
            pl.BlockSpec(memory_space=pltpu.MemorySpace.HBM),
        ],
        out_specs=[
            pl.BlockSpec((B, P), lambda i: (0, 0)),
            pl.BlockSpec((B * CV, P), lambda i: (0, 0)),
            pl.BlockSpec(memory_space=pltpu.MemorySpace.HBM),
        ],
        out_shape=[
            jax.ShapeDtypeStruct((B, P), jnp.int32),
            jax.ShapeDtypeStruct((B * CV, P), jnp.float32),
            jax.ShapeDtypeStruct((R, P), jnp.float32),
        ],
        scratch_shapes=[
            pltpu.SMEM((1,), jnp.int32),
            pltpu.VMEM((2, R, _BLK), jnp.float32),
            pltpu.VMEM((2, R, _BLK), jnp.float32),
            pltpu.VMEM((R, _BLK), jnp.float32),
            pltpu.SemaphoreType.DMA((2,)),
            pltpu.SemaphoreType.DMA((2,)),
        ],
        compiler_params=pltpu.CompilerParams(
            dimension_semantics=("arbitrary",),
        ),
    )(mr, vr, xr)
    return (xo.reshape(B, C, P), vo.reshape(B, CV, P), mo.reshape(B, 1, P))


# speedup vs baseline: 1.0685x; 1.0685x over previous
import functools

import jax
import jax.numpy as jnp
from jax.experimental import pallas as pl
from jax.experimental.pallas import tpu as pltpu


_BLK = 256
_NIN = 3


def _trim_body(nblk, m_ref, v_ref, x_hbm, mo_ref, vo_ref, xo_hbm,
               ml_s, xin, xout, zbuf, in_sem, out_sem, zsem):
    i = pl.program_id(0)

    @pl.when(i == 0)
    def _prologue():
        pltpu.make_async_copy(
            x_hbm.at[:, pl.ds(0, _BLK)], xin.at[0], in_sem.at[0]).start()
        counts = jnp.sum((m_ref[...] != 0).astype(jnp.int32), axis=1)
        ml0 = jnp.maximum(jnp.max(counts), 1)
        ml_s[0] = ml0

        @pl.when(_BLK < ml0)
        def _():
            pltpu.make_async_copy(
                x_hbm.at[:, pl.ds(_BLK, _BLK)], xin.at[1], in_sem.at[1]).start()

        col = jax.lax.broadcasted_iota(jnp.int32, (1, v_ref.shape[1]), 1)
        keep = col < ml0
        vo_ref[...] = jnp.where(keep, v_ref[...], 0.0)
        mo_ref[...] = jnp.where(keep & (m_ref[...] != 0), 1, 0).astype(jnp.int32)
        zbuf[...] = jnp.zeros_like(zbuf)

    ml = ml_s[0]
    jlast = (ml - 1) // _BLK

    oslot = jax.lax.rem(i, 2)
    prev = i - 2

    @pl.when((prev >= 0) & (prev <= jlast))
    def _wait_prev_out():
        pltpu.make_async_copy(
            xout.at[oslot], xo_hbm.at[:, pl.ds(prev * _BLK, _BLK)],
            out_sem.at[oslot]).wait()

    nxt = i + 2

    @pl.when((nxt < nblk) & (nxt <= jlast))
    def _start_next():
        pltpu.make_async_copy(
            x_hbm.at[:, pl.ds(nxt * _BLK, _BLK)],
            xin.at[jax.lax.rem(nxt, _NIN)],
            in_sem.at[jax.lax.rem(nxt, _NIN)]).start()

    bz = jlast + 1 + i

    @pl.when(bz < nblk)
    def _zero_write():
        pltpu.make_async_copy(
            zbuf, xo_hbm.at[:, pl.ds(bz * _BLK, _BLK)], zsem).start()

    @pl.when(i <= jlast)
    def _compute():
        islot = jax.lax.rem(i, _NIN)
        pltpu.make_async_copy(
            x_hbm.at[:, pl.ds(i * _BLK, _BLK)], xin.at[islot],
            in_sem.at[islot]).wait()
        col = i * _BLK + jax.lax.broadcasted_iota(jnp.int32, (1, _BLK), 1)
        xout[oslot] = jnp.where(col < ml, xin[islot], 0.0)
        pltpu.make_async_copy(
            xout.at[oslot], xo_hbm.at[:, pl.ds(i * _BLK, _BLK)],
            out_sem.at[oslot]).start()

    @pl.when(i == nblk - 1)
    def _epilogue():
        for d in (nblk - 2, nblk - 1):
            s = d % 2

            @pl.when(d <= jlast)
            def _():
                pltpu.make_async_copy(
                    xout.at[s], xo_hbm.at[:, pl.ds(d * _BLK, _BLK)],
                    out_sem.at[s]).wait()

        for d in range(1, nblk):
            @pl.when(d > jlast)
            def _():
                pltpu.make_async_copy(
                    zbuf, xo_hbm.at[:, pl.ds(d * _BLK, _BLK)], zsem).wait()


def kernel(x, v, mask):
    B, C, P = x.shape
    CV = v.shape[1]
    R = B * C
    nblk = P // _BLK
    xr = x.reshape(R, P)
    vr = v.reshape(B * CV, P)
    mr = mask.reshape(B, P)

    body = functools.partial(_trim_body, nblk)

    mo, vo, xo = pl.pallas_call(
        body,
        grid=(nblk,),
        in_specs=[
            pl.BlockSpec((B, P), lambda i: (0, 0)),
            pl.BlockSpec((B * CV, P), lambda i: (0, 0)),
            pl.BlockSpec(memory_space=pltpu.MemorySpace.HBM),
        ],
        out_specs=[
            pl.BlockSpec((B, P), lambda i: (0, 0)),
            pl.BlockSpec((B * CV, P), lambda i: (0, 0)),
            pl.BlockSpec(memory_space=pltpu.MemorySpace.HBM),
        ],
        out_shape=[
            jax.ShapeDtypeStruct((B, P), jnp.int32),
            jax.ShapeDtypeStruct((B * CV, P), jnp.float32),
            jax.ShapeDtypeStruct((R, P), jnp.float32),
        ],
        scratch_shapes=[
            pltpu.SMEM((1,), jnp.int32),
            pltpu.VMEM((_NIN, R, _BLK), jnp.float32),
            pltpu.VMEM((2, R, _BLK), jnp.float32),
            pltpu.VMEM((R, _BLK), jnp.float32),
            pltpu.SemaphoreType.DMA((_NIN,)),
            pltpu.SemaphoreType.DMA((2,)),
            pltpu.SemaphoreType.DMA,
        ],
        compiler_params=pltpu.CompilerParams(
            dimension_semantics=("arbitrary",),
        ),
    )(mr, vr, xr)
    return (xo.reshape(B, C, P), vo.reshape(B, CV, P), mo.reshape(B, 1, P))


# speedup vs baseline: 1.0863x; 1.0166x over previous
import functools

import jax
import jax.numpy as jnp
from jax.experimental import pallas as pl
from jax.experimental.pallas import tpu as pltpu


_BLK = 256
_NIN = 3


def _trim_body(nblk, m_hbm, v_hbm, x_hbm, mo_hbm, vo_hbm, xo_hbm,
               ml_s, m_v, v_v, mo_v, vo_v, xin, xout, zbuf,
               msem, vsem, vosem, in_sem, out_sem, zsem):
    i = pl.program_id(0)

    @pl.when(i == 0)
    def _prologue():
        pltpu.make_async_copy(
            x_hbm.at[:, pl.ds(0, _BLK)], xin.at[0], in_sem.at[0]).start()
        pltpu.make_async_copy(m_hbm, m_v, msem).start()
        pltpu.make_async_copy(v_hbm, v_v, vsem).start()
        pltpu.make_async_copy(m_hbm, m_v, msem).wait()
        counts = jnp.sum((m_v[...] != 0).astype(jnp.int32), axis=1)
        ml0 = jnp.maximum(jnp.max(counts), 1)
        ml_s[0] = ml0

        @pl.when(_BLK < ml0)
        def _():
            pltpu.make_async_copy(
                x_hbm.at[:, pl.ds(_BLK, _BLK)], xin.at[1], in_sem.at[1]).start()

        zbuf[...] = jnp.zeros_like(zbuf)

    ml = ml_s[0]
    jlast = (ml - 1) // _BLK

    @pl.when(i == 1)
    def _small_tensors():
        col = jax.lax.broadcasted_iota(jnp.int32, (1, v_v.shape[1]), 1)
        keep = col < ml
        pltpu.make_async_copy(v_hbm, v_v, vsem).wait()
        vo_v[...] = jnp.where(keep, v_v[...], 0.0)
        mo_v[...] = jnp.where(keep & (m_v[...] != 0), 1, 0).astype(jnp.int32)
        pltpu.make_async_copy(vo_v, vo_hbm, vosem).start()
        pltpu.make_async_copy(mo_v, mo_hbm, vosem).start()

    oslot = jax.lax.rem(i, 2)
    prev = i - 2

    @pl.when((prev >= 0) & (prev <= jlast))
    def _wait_prev_out():
        pltpu.make_async_copy(
            xout.at[oslot], xo_hbm.at[:, pl.ds(prev * _BLK, _BLK)],
            out_sem.at[oslot]).wait()

    nxt = i + 2

    @pl.when((nxt < nblk) & (nxt <= jlast))
    def _start_next():
        pltpu.make_async_copy(
            x_hbm.at[:, pl.ds(nxt * _BLK, _BLK)],
            xin.at[jax.lax.rem(nxt, _NIN)],
            in_sem.at[jax.lax.rem(nxt, _NIN)]).start()

    bz = jlast + 1 + i

    @pl.when(bz < nblk)
    def _zero_write():
        pltpu.make_async_copy(
            zbuf, xo_hbm.at[:, pl.ds(bz * _BLK, _BLK)], zsem).start()

    @pl.when(i <= jlast)
    def _compute():
        islot = jax.lax.rem(i, _NIN)
        pltpu.make_async_copy(
            x_hbm.at[:, pl.ds(i * _BLK, _BLK)], xin.at[islot],
            in_sem.at[islot]).wait()
        col = i * _BLK + jax.lax.broadcasted_iota(jnp.int32, (1, _BLK), 1)
        xout[oslot] = jnp.where(col < ml, xin[islot], 0.0)
        pltpu.make_async_copy(
            xout.at[oslot], xo_hbm.at[:, pl.ds(i * _BLK, _BLK)],
            out_sem.at[oslot]).start()

    @pl.when(i == nblk - 1)
    def _epilogue():
        for d in (nblk - 2, nblk - 1):
            s = d % 2

            @pl.when(d <= jlast)
            def _():
                pltpu.make_async_copy(
                    xout.at[s], xo_hbm.at[:, pl.ds(d * _BLK, _BLK)],
                    out_sem.at[s]).wait()

        for d in range(1, nblk):
            @pl.when(d > jlast)
            def _():
                pltpu.make_async_copy(
                    zbuf, xo_hbm.at[:, pl.ds(d * _BLK, _BLK)], zsem).wait()

        pltpu.make_async_copy(vo_v, vo_hbm, vosem).wait()
        pltpu.make_async_copy(mo_v, mo_hbm, vosem).wait()


def kernel(x, v, mask):
    B, C, P = x.shape
    CV = v.shape[1]
    R = B * C
    nblk = P // _BLK
    xr = x.reshape(R, P)
    vr = v.reshape(B * CV, P)
    mr = mask.reshape(B, P)

    body = functools.partial(_trim_body, nblk)
    hbm = pl.BlockSpec(memory_space=pltpu.MemorySpace.HBM)

    mo, vo, xo = pl.pallas_call(
        body,
        grid=(nblk,),
        in_specs=[hbm, hbm, hbm],
        out_specs=[hbm, hbm, hbm],
        out_shape=[
            jax.ShapeDtypeStruct((B, P), jnp.int32),
            jax.ShapeDtypeStruct((B * CV, P), jnp.float32),
            jax.ShapeDtypeStruct((R, P), jnp.float32),
        ],
        scratch_shapes=[
            pltpu.SMEM((1,), jnp.int32),
            pltpu.VMEM((B, P), jnp.int32),
            pltpu.VMEM((B * CV, P), jnp.float32),
            pltpu.VMEM((B, P), jnp.int32),
            pltpu.VMEM((B * CV, P), jnp.float32),
            pltpu.VMEM((_NIN, R, _BLK), jnp.float32),
            pltpu.VMEM((2, R, _BLK), jnp.float32),
            pltpu.VMEM((R, _BLK), jnp.float32),
            pltpu.SemaphoreType.DMA,
            pltpu.SemaphoreType.DMA,
            pltpu.SemaphoreType.DMA,
            pltpu.SemaphoreType.DMA((_NIN,)),
            pltpu.SemaphoreType.DMA((2,)),
            pltpu.SemaphoreType.DMA,
        ],
        compiler_params=pltpu.CompilerParams(
            dimension_semantics=("arbitrary",),
        ),
    )(mr, vr, xr)
    return (xo.reshape(B, C, P), vo.reshape(B, CV, P), mo.reshape(B, 1, P))


# speedup vs baseline: 1.1113x; 1.0231x over previous
import functools

import jax
import jax.numpy as jnp
from jax.experimental import pallas as pl
from jax.experimental.pallas import tpu as pltpu


_BLK = 512
_NIN = 3


def _trim_body(nblk, m_hbm, v_hbm, x_hbm, mo_hbm, vo_hbm, xo_hbm,
               ml_s, m_v, v_v, mo_v, vo_v, xin, xout, zbuf,
               msem, vsem, vosem, in_sem, out_sem, zsem):
    i = pl.program_id(0)

    @pl.when(i == 0)
    def _prologue():
        pltpu.make_async_copy(
            x_hbm.at[:, pl.ds(0, _BLK)], xin.at[0], in_sem.at[0]).start()
        pltpu.make_async_copy(m_hbm, m_v, msem).start()
        pltpu.make_async_copy(v_hbm, v_v, vsem).start()
        pltpu.make_async_copy(m_hbm, m_v, msem).wait()
        counts = jnp.sum((m_v[...] != 0).astype(jnp.int32), axis=1)
        ml0 = jnp.maximum(jnp.max(counts), 1)
        ml_s[0] = ml0

        @pl.when(_BLK < ml0)
        def _():
            pltpu.make_async_copy(
                x_hbm.at[:, pl.ds(_BLK, _BLK)], xin.at[1], in_sem.at[1]).start()

        zbuf[...] = jnp.zeros_like(zbuf)

    ml = ml_s[0]
    jlast = (ml - 1) // _BLK

    @pl.when(i == 1)
    def _small_tensors():
        col = jax.lax.broadcasted_iota(jnp.int32, (1, v_v.shape[1]), 1)
        keep = col < ml
        pltpu.make_async_copy(v_hbm, v_v, vsem).wait()
        vo_v[...] = jnp.where(keep, v_v[...], 0.0)
        mo_v[...] = jnp.where(keep & (m_v[...] != 0), 1, 0).astype(jnp.int32)
        pltpu.make_async_copy(vo_v, vo_hbm, vosem).start()
        pltpu.make_async_copy(mo_v, mo_hbm, vosem).start()

    oslot = jax.lax.rem(i, 2)
    prev = i - 2

    @pl.when((prev >= 0) & (prev <= jlast))
    def _wait_prev_out():
        pltpu.make_async_copy(
            xout.at[oslot], xo_hbm.at[:, pl.ds(prev * _BLK, _BLK)],
            out_sem.at[oslot]).wait()

    nxt = i + 2

    @pl.when((nxt < nblk) & (nxt <= jlast))
    def _start_next():
        pltpu.make_async_copy(
            x_hbm.at[:, pl.ds(nxt * _BLK, _BLK)],
            xin.at[jax.lax.rem(nxt, _NIN)],
            in_sem.at[jax.lax.rem(nxt, _NIN)]).start()

    bz = jlast + 1 + i

    @pl.when(bz < nblk)
    def _zero_write():
        pltpu.make_async_copy(
            zbuf, xo_hbm.at[:, pl.ds(bz * _BLK, _BLK)], zsem).start()

    @pl.when(i <= jlast)
    def _compute():
        islot = jax.lax.rem(i, _NIN)
        pltpu.make_async_copy(
            x_hbm.at[:, pl.ds(i * _BLK, _BLK)], xin.at[islot],
            in_sem.at[islot]).wait()
        col = i * _BLK + jax.lax.broadcasted_iota(jnp.int32, (1, _BLK), 1)
        xout[oslot] = jnp.where(col < ml, xin[islot], 0.0)
        pltpu.make_async_copy(
            xout.at[oslot], xo_hbm.at[:, pl.ds(i * _BLK, _BLK)],
            out_sem.at[oslot]).start()

    @pl.when(i == nblk - 1)
    def _epilogue():
        for d in (nblk - 2, nblk - 1):
            s = d % 2

            @pl.when(d <= jlast)
            def _():
                pltpu.make_async_copy(
                    xout.at[s], xo_hbm.at[:, pl.ds(d * _BLK, _BLK)],
                    out_sem.at[s]).wait()

        for d in range(1, nblk):
            @pl.when(d > jlast)
            def _():
                pltpu.make_async_copy(
                    zbuf, xo_hbm.at[:, pl.ds(d * _BLK, _BLK)], zsem).wait()

        pltpu.make_async_copy(vo_v, vo_hbm, vosem).wait()
        pltpu.make_async_copy(mo_v, mo_hbm, vosem).wait()


def kernel(x, v, mask):
    B, C, P = x.shape
    CV = v.shape[1]
    R = B * C
    nblk = P // _BLK
    xr = x.reshape(R, P)
    vr = v.reshape(B * CV, P)
    mr = mask.reshape(B, P)

    body = functools.partial(_trim_body, nblk)
    hbm = pl.BlockSpec(memory_space=pltpu.MemorySpace.HBM)

    mo, vo, xo = pl.pallas_call(
        body,
        grid=(nblk,),
        in_specs=[hbm, hbm, hbm],
        out_specs=[hbm, hbm, hbm],
        out_shape=[
            jax.ShapeDtypeStruct((B, P), jnp.int32),
            jax.ShapeDtypeStruct((B * CV, P), jnp.float32),
            jax.ShapeDtypeStruct((R, P), jnp.float32),
        ],
        scratch_shapes=[
            pltpu.SMEM((1,), jnp.int32),
            pltpu.VMEM((B, P), jnp.int32),
            pltpu.VMEM((B * CV, P), jnp.float32),
            pltpu.VMEM((B, P), jnp.int32),
            pltpu.VMEM((B * CV, P), jnp.float32),
            pltpu.VMEM((_NIN, R, _BLK), jnp.float32),
            pltpu.VMEM((2, R, _BLK), jnp.float32),
            pltpu.VMEM((R, _BLK), jnp.float32),
            pltpu.SemaphoreType.DMA,
            pltpu.SemaphoreType.DMA,
            pltpu.SemaphoreType.DMA,
            pltpu.SemaphoreType.DMA((_NIN,)),
            pltpu.SemaphoreType.DMA((2,)),
            pltpu.SemaphoreType.DMA,
        ],
        compiler_params=pltpu.CompilerParams(
            dimension_semantics=("arbitrary",),
        ),
    )(mr, vr, xr)
    return (xo.reshape(B, C, P), vo.reshape(B, CV, P), mo.reshape(B, 1, P))
